# Initial kernel scaffold; baseline (speedup 1.0000x reference)
import jax
import jax.numpy as jnp
from jax.experimental import pallas as pl


def kernel(x, edge_indices, batch, params):
    raise NotImplementedError("write your pallas kernel here")


# speedup vs baseline: 5.1779x; 5.1779x over previous
import functools

import jax
import jax.numpy as jnp
from jax import lax
from jax.experimental import pallas as pl
from jax.experimental.pallas import tpu as pltpu
from jax.experimental.pallas import tpu_sc as plsc

N = 10000
D_IN = 128
E = 160000
M = 13
HID = 64
CD = 6
DIM = 64
G = 128
OUT = 10

NC = 2
NS = 16
NW = NC * NS
CH = 128
EPT = E // NW
EPT_PAD = 5120
NCHUNK = EPT_PAD // CH
ROWS_T = 624
ROWS_LAST = 640
ZR = 208
HIDP = 128

BN1 = 2000
BN = 1000
NB = N // BN


def _sc_body(v_hbm, src_hbm, dst_hbm, out_hbm,
             src_v, dst_v, srcc_v, dstc_v, rows_v, zeros_v, acc_sh, gsem):
    c = lax.axis_index("c")
    s = lax.axis_index("s")
    w = c * NS + s

    base = s * ROWS_T

    def _zero_row(r, carry):
        for j in range(HID // 16):
            zeros_v[r, pl.ds(j * 16, 16)] = jnp.zeros((16,), jnp.float32)
        return carry
    lax.fori_loop(0, ZR, _zero_row, 0)

    def _motif(m, carry):
        for z in range(ROWS_T // ZR):
            pltpu.sync_copy(zeros_v, acc_sh.at[pl.ds(base + z * ZR, ZR)])

        @pl.when(s == NS - 1)
        def _():
            pltpu.sync_copy(zeros_v.at[pl.ds(0, ROWS_LAST - ROWS_T)],
                            acc_sh.at[pl.ds(base + ROWS_T, ROWS_LAST - ROWS_T)])
        plsc.subcore_barrier()

        pltpu.sync_copy(src_hbm.at[m, w], src_v)
        pltpu.sync_copy(dst_hbm.at[m, w], dst_v)

        def _chunk(i, carry2):
            for j in range(CH // 16):
                srcc_v[pl.ds(j * 16, 16)] = src_v[i, pl.ds(j * 16, 16)]
                dstc_v[pl.ds(j * 16, 16)] = dst_v[i, pl.ds(j * 16, 16)]
            pltpu.async_copy(v_hbm.at[srcc_v], rows_v, gsem).wait()
            pltpu.sync_copy(rows_v, acc_sh.at[dstc_v], add=True)
            return carry2
        lax.fori_loop(0, NCHUNK, _chunk, 0)

        plsc.subcore_barrier()

        @pl.when(s < NS - 1)
        def _():
            pltpu.sync_copy(acc_sh.at[pl.ds(base, ROWS_T)],
                            out_hbm.at[c, m, pl.ds(base, ROWS_T)])

        @pl.when(s == NS - 1)
        def _():
            pltpu.sync_copy(acc_sh.at[pl.ds(base, ROWS_LAST)],
                            out_hbm.at[c, m, pl.ds(base, ROWS_LAST)])
        return carry
    lax.fori_loop(0, M, _motif, 0)


def _sc_segsum(v_flat, src_p, dst_p):
    mesh = plsc.VectorSubcoreMesh(core_axis_name="c", subcore_axis_name="s",
                                  num_cores=NC, num_subcores=NS)
    kern = functools.partial(
        pl.kernel,
        out_type=jax.ShapeDtypeStruct((NC, M, N, HID), jnp.float32),
        mesh=mesh,
        scratch_types=[
            pltpu.VMEM((NCHUNK, CH), jnp.int32),
            pltpu.VMEM((NCHUNK, CH), jnp.int32),
            pltpu.VMEM((CH,), jnp.int32),
            pltpu.VMEM((CH,), jnp.int32),
            pltpu.VMEM((CH, HID), jnp.float32),
            pltpu.VMEM((ZR, HID), jnp.float32),
            pltpu.VMEM_SHARED((N + 16, HID), jnp.float32),
            pltpu.SemaphoreType.DMA,
        ],
        compiler_params=pltpu.CompilerParams(use_tc_tiling_on_sc=False),
    )(_sc_body)
    return kern(v_flat, src_p, dst_p)


def _tc_first_body(x_ref, w_ref, a_ref, v_ref):
    xb = x_ref[...]
    for m in range(M):
        hw = jnp.dot(xb, w_ref[m], preferred_element_type=jnp.float32)
        sc = jnp.tanh(jnp.dot(hw, a_ref[m], preferred_element_type=jnp.float32))
        v_ref[m] = hw * sc


def _h_from_partials(p_ref, cw1_ref, b1_ref, w2_ref, b2_ref):
    acc = jnp.broadcast_to(b1_ref[...], (p_ref.shape[2], DIM))
    for m in range(M):
        t = jnp.tanh(p_ref[0, m] + p_ref[1, m])
        acc = acc + jnp.dot(t, cw1_ref[m], preferred_element_type=jnp.float32)
    h = jnp.maximum(acc, 0.0)
    h = jnp.dot(h, w2_ref[...], preferred_element_type=jnp.float32) + b2_ref[...]
    return jnp.maximum(h, 0.0)


def _tc_mid_body(p_ref, cw1_ref, b1_ref, w2_ref, b2_ref, wn_ref, an_ref, v_ref):
    h = _h_from_partials(p_ref, cw1_ref, b1_ref, w2_ref, b2_ref)
    for m in range(M):
        hw = jnp.dot(h, wn_ref[m], preferred_element_type=jnp.float32)
        sc = jnp.tanh(jnp.dot(hw, an_ref[m], preferred_element_type=jnp.float32))
        v_ref[m] = hw * sc


def _tc_final_body(p_ref, cw1_ref, b1_ref, w2_ref, b2_ref, batch_ref,
                   l1_ref, l1b_ref, l2_ref, l2b_ref, out_ref, hg_acc):
    i = pl.program_id(0)
    h = _h_from_partials(p_ref, cw1_ref, b1_ref, w2_ref, b2_ref)
    b = batch_ref[0]
    oh = (lax.broadcasted_iota(jnp.int32, (G, BN), 0) == b).astype(jnp.float32)
    contrib = jnp.dot(oh, h, preferred_element_type=jnp.float32)

    @pl.when(i == 0)
    def _():
        hg_acc[...] = contrib

    @pl.when(i > 0)
    def _():
        hg_acc[...] = hg_acc[...] + contrib

    @pl.when(i == NB - 1)
    def _():
        hg = jnp.maximum(
            jnp.dot(hg_acc[...], l1_ref[...], preferred_element_type=jnp.float32)
            + l1b_ref[...], 0.0)
        logits = jnp.dot(hg, l2_ref[...], preferred_element_type=jnp.float32) + l2b_ref[...]
        mx = jnp.max(logits, axis=1, keepdims=True)
        z = logits - mx
        lse = jnp.log(jnp.sum(jnp.exp(z), axis=1, keepdims=True))
        out_ref[...] = z - lse


def _tc_first(x, w0, a0):
    nb = N // BN1
    return pl.pallas_call(
        _tc_first_body,
        grid=(nb,),
        in_specs=[
            pl.BlockSpec((BN1, D_IN), lambda i: (i, 0)),
            pl.BlockSpec((M, D_IN, HID), lambda i: (0, 0, 0)),
            pl.BlockSpec((M, HID, 1), lambda i: (0, 0, 0)),
        ],
        out_specs=pl.BlockSpec((M, BN1, HID), lambda i: (0, i, 0)),
        out_shape=jax.ShapeDtypeStruct((M, N, HID), jnp.float32),
    )(x, w0, a0)


def _tc_mid(part, cw1, b1, w2, b2, wn, an):
    return pl.pallas_call(
        _tc_mid_body,
        grid=(NB,),
        in_specs=[
            pl.BlockSpec((NC, M, BN, HID), lambda i: (0, 0, i, 0)),
            pl.BlockSpec((M, HID, DIM), lambda i: (0, 0, 0)),
            pl.BlockSpec((1, DIM), lambda i: (0, 0)),
            pl.BlockSpec((DIM, DIM), lambda i: (0, 0)),
            pl.BlockSpec((1, DIM), lambda i: (0, 0)),
            pl.BlockSpec((M, DIM, HID), lambda i: (0, 0, 0)),
            pl.BlockSpec((M, HID, 1), lambda i: (0, 0, 0)),
        ],
        out_specs=pl.BlockSpec((M, BN, HID), lambda i: (0, i, 0)),
        out_shape=jax.ShapeDtypeStruct((M, N, HID), jnp.float32),
    )(part, cw1, b1, w2, b2, wn, an)


def _tc_final(part, cw1, b1, w2, b2, batch3, l1, l1b, l2, l2b):
    return pl.pallas_call(
        _tc_final_body,
        grid=(NB,),
        in_specs=[
            pl.BlockSpec((NC, M, BN, HID), lambda i: (0, 0, i, 0)),
            pl.BlockSpec((M, HID, DIM), lambda i: (0, 0, 0)),
            pl.BlockSpec((1, DIM), lambda i: (0, 0)),
            pl.BlockSpec((DIM, DIM), lambda i: (0, 0)),
            pl.BlockSpec((1, DIM), lambda i: (0, 0)),
            pl.BlockSpec((1, 1, BN), lambda i: (i, 0, 0)),
            pl.BlockSpec((DIM, DIM), lambda i: (0, 0)),
            pl.BlockSpec((1, DIM), lambda i: (0, 0)),
            pl.BlockSpec((DIM, OUT), lambda i: (0, 0)),
            pl.BlockSpec((1, OUT), lambda i: (0, 0)),
        ],
        out_specs=pl.BlockSpec((G, OUT), lambda i: (0, 0)),
        out_shape=jax.ShapeDtypeStruct((G, OUT), jnp.float32),
        scratch_shapes=[pltpu.VMEM((G, DIM), jnp.float32)],
    )(part, cw1, b1, w2, b2, batch3, l1, l1b, l2, l2b)


def _prep_mlp(params, l):
    inv = 1.0 / jnp.sqrt(1.0 + 1e-5)
    g = params['bn%d_g' % l] * inv
    w1 = params['mlp%d_w1' % l] * g[None, :]
    b1 = params['mlp%d_b1' % l] * g + params['bn%d_b' % l]
    cw1 = jnp.einsum('mhc,mcd->mhd', params['C%d' % l],
                     w1.reshape(M, CD, DIM))
    return cw1, b1[None, :]


def kernel(x, edge_indices, batch, params):
    src = edge_indices[:, 0, :].astype(jnp.int32)
    dst = edge_indices[:, 1, :].astype(jnp.int32)
    src_adj = src + (jnp.arange(M, dtype=jnp.int32) * N)[:, None]
    src_p = jnp.pad(src_adj.reshape(M, NW, EPT),
                    ((0, 0), (0, 0), (0, EPT_PAD - EPT))).reshape(M, NW, NCHUNK, CH)
    dst_p = jnp.pad(dst.reshape(M, NW, EPT),
                    ((0, 0), (0, 0), (0, EPT_PAD - EPT)),
                    constant_values=N).reshape(M, NW, NCHUNK, CH)

    batch3 = batch.astype(jnp.int32).reshape(NB, 1, BN)
    l1, l1b = params['lin1_w'], params['lin1_b'][None, :]
    l2, l2b = params['lin2_w'], params['lin2_b'][None, :]

    v = _tc_first(x, params['W0'], params['a0'][:, :, None])
    for l in range(3):
        part = _sc_segsum(v.reshape(M * N, HID), src_p, dst_p)
        cw1, b1 = _prep_mlp(params, l)
        w2 = params['mlp%d_w2' % l]
        b2 = params['mlp%d_b2' % l][None, :]
        if l < 2:
            v = _tc_mid(part, cw1, b1, w2, b2,
                        params['W%d' % (l + 1)], params['a%d' % (l + 1)][:, :, None])
        else:
            return _tc_final(part, cw1, b1, w2, b2, batch3, l1, l1b, l2, l2b)


# speedup vs baseline: 6.4940x; 1.2542x over previous
import functools

import jax
import jax.numpy as jnp
from jax import lax
from jax.experimental import pallas as pl
from jax.experimental.pallas import tpu as pltpu
from jax.experimental.pallas import tpu_sc as plsc

N = 10000
D_IN = 128
E = 160000
M = 13
HID = 64
CD = 6
DIM = 64
G = 128
OUT = 10

NC = 2
NS = 16
NW = NC * NS
CH = 128
EPT = E // NW
EPT_PAD = 5120
NCHUNK = EPT_PAD // CH
ROWS_T = 624
ROWS_LAST = 640
ZR = 208
HIDP = 128

BN1 = 2000
BN = 1000
NB = N // BN


NBUF = 4
DEPTH = 2


def _sc_body(v_hbm, src_hbm, dst_hbm, out_hbm,
             src_v, dst_v, rows_v, zeros_v, acc_sh, gsem, ssem):
    c = lax.axis_index("c")
    s = lax.axis_index("s")
    w = c * NS + s

    base = s * ROWS_T

    def _zero_row(r, carry):
        for j in range(HID // 16):
            zeros_v[r, pl.ds(j * 16, 16)] = jnp.zeros((16,), jnp.float32)
        return carry
    lax.fori_loop(0, ZR, _zero_row, 0)

    def _motif(m, carry):
        for z in range(ROWS_T // ZR):
            pltpu.sync_copy(zeros_v, acc_sh.at[pl.ds(base + z * ZR, ZR)])

        @pl.when(s == NS - 1)
        def _():
            pltpu.sync_copy(zeros_v.at[pl.ds(0, ROWS_LAST - ROWS_T)],
                            acc_sh.at[pl.ds(base + ROWS_T, ROWS_LAST - ROWS_T)])
        plsc.subcore_barrier()

        pltpu.sync_copy(src_hbm.at[m, w], src_v)
        pltpu.sync_copy(dst_hbm.at[m, w], dst_v)

        for p in range(DEPTH):
            pltpu.async_copy(v_hbm.at[src_v.at[p]], rows_v.at[p], gsem)

        def _chunk(i, carry2):
            slot = lax.rem(i, NBUF)

            @pl.when(i >= 2)
            def _():
                sl = lax.rem(i - 2, NBUF)
                pltpu.make_async_copy(rows_v.at[sl],
                                      acc_sh.at[dst_v.at[i - 2]], ssem).wait()

            @pl.when(i + DEPTH < NCHUNK)
            def _():
                sl = lax.rem(i + DEPTH, NBUF)
                pltpu.async_copy(v_hbm.at[src_v.at[i + DEPTH]],
                                 rows_v.at[sl], gsem)

            pltpu.make_async_copy(v_hbm.at[src_v.at[i]],
                                  rows_v.at[slot], gsem).wait()
            pltpu.async_copy(rows_v.at[slot], acc_sh.at[dst_v.at[i]],
                             ssem, add=True)
            return carry2
        lax.fori_loop(0, NCHUNK, _chunk, 0)

        for p in range(2):
            pltpu.make_async_copy(rows_v.at[p],
                                  acc_sh.at[dst_v.at[p]], ssem).wait()

        plsc.subcore_barrier()

        @pl.when(s < NS - 1)
        def _():
            pltpu.sync_copy(acc_sh.at[pl.ds(base, ROWS_T)],
                            out_hbm.at[c, m, pl.ds(base, ROWS_T)])

        @pl.when(s == NS - 1)
        def _():
            pltpu.sync_copy(acc_sh.at[pl.ds(base, ROWS_LAST)],
                            out_hbm.at[c, m, pl.ds(base, ROWS_LAST)])
        return carry
    lax.fori_loop(0, M, _motif, 0)


def _sc_segsum(v_flat, src_p, dst_p):
    mesh = plsc.VectorSubcoreMesh(core_axis_name="c", subcore_axis_name="s",
                                  num_cores=NC, num_subcores=NS)
    kern = functools.partial(
        pl.kernel,
        out_type=jax.ShapeDtypeStruct((NC, M, N, HID), jnp.float32),
        mesh=mesh,
        scratch_types=[
            pltpu.VMEM((NCHUNK, CH), jnp.int32),
            pltpu.VMEM((NCHUNK, CH), jnp.int32),
            pltpu.VMEM((NBUF, CH, HID), jnp.float32),
            pltpu.VMEM((ZR, HID), jnp.float32),
            pltpu.VMEM_SHARED((N + 16, HID), jnp.float32),
            pltpu.SemaphoreType.DMA,
            pltpu.SemaphoreType.DMA,
        ],
        compiler_params=pltpu.CompilerParams(use_tc_tiling_on_sc=False),
    )(_sc_body)
    return kern(v_flat, src_p, dst_p)


def _tc_first_body(x_ref, w_ref, a_ref, v_ref):
    xb = x_ref[...]
    for m in range(M):
        hw = jnp.dot(xb, w_ref[m], preferred_element_type=jnp.float32)
        sc = jnp.tanh(jnp.dot(hw, a_ref[m], preferred_element_type=jnp.float32))
        v_ref[m] = hw * sc


def _h_from_partials(p_ref, cw1_ref, b1_ref, w2_ref, b2_ref):
    acc = jnp.broadcast_to(b1_ref[...], (p_ref.shape[2], DIM))
    for m in range(M):
        t = jnp.tanh(p_ref[0, m] + p_ref[1, m])
        acc = acc + jnp.dot(t, cw1_ref[m], preferred_element_type=jnp.float32)
    h = jnp.maximum(acc, 0.0)
    h = jnp.dot(h, w2_ref[...], preferred_element_type=jnp.float32) + b2_ref[...]
    return jnp.maximum(h, 0.0)


def _tc_mid_body(p_ref, cw1_ref, b1_ref, w2_ref, b2_ref, wn_ref, an_ref, v_ref):
    h = _h_from_partials(p_ref, cw1_ref, b1_ref, w2_ref, b2_ref)
    for m in range(M):
        hw = jnp.dot(h, wn_ref[m], preferred_element_type=jnp.float32)
        sc = jnp.tanh(jnp.dot(hw, an_ref[m], preferred_element_type=jnp.float32))
        v_ref[m] = hw * sc


def _tc_final_body(p_ref, cw1_ref, b1_ref, w2_ref, b2_ref, batch_ref,
                   l1_ref, l1b_ref, l2_ref, l2b_ref, out_ref, hg_acc):
    i = pl.program_id(0)
    h = _h_from_partials(p_ref, cw1_ref, b1_ref, w2_ref, b2_ref)
    b = batch_ref[0]
    oh = (lax.broadcasted_iota(jnp.int32, (G, BN), 0) == b).astype(jnp.float32)
    contrib = jnp.dot(oh, h, preferred_element_type=jnp.float32)

    @pl.when(i == 0)
    def _():
        hg_acc[...] = contrib

    @pl.when(i > 0)
    def _():
        hg_acc[...] = hg_acc[...] + contrib

    @pl.when(i == NB - 1)
    def _():
        hg = jnp.maximum(
            jnp.dot(hg_acc[...], l1_ref[...], preferred_element_type=jnp.float32)
            + l1b_ref[...], 0.0)
        logits = jnp.dot(hg, l2_ref[...], preferred_element_type=jnp.float32) + l2b_ref[...]
        mx = jnp.max(logits, axis=1, keepdims=True)
        z = logits - mx
        lse = jnp.log(jnp.sum(jnp.exp(z), axis=1, keepdims=True))
        out_ref[...] = z - lse


def _tc_first(x, w0, a0):
    nb = N // BN1
    return pl.pallas_call(
        _tc_first_body,
        grid=(nb,),
        in_specs=[
            pl.BlockSpec((BN1, D_IN), lambda i: (i, 0)),
            pl.BlockSpec((M, D_IN, HID), lambda i: (0, 0, 0)),
            pl.BlockSpec((M, HID, 1), lambda i: (0, 0, 0)),
        ],
        out_specs=pl.BlockSpec((M, BN1, HID), lambda i: (0, i, 0)),
        out_shape=jax.ShapeDtypeStruct((M, N, HID), jnp.float32),
    )(x, w0, a0)


def _tc_mid(part, cw1, b1, w2, b2, wn, an):
    return pl.pallas_call(
        _tc_mid_body,
        grid=(NB,),
        in_specs=[
            pl.BlockSpec((NC, M, BN, HID), lambda i: (0, 0, i, 0)),
            pl.BlockSpec((M, HID, DIM), lambda i: (0, 0, 0)),
            pl.BlockSpec((1, DIM), lambda i: (0, 0)),
            pl.BlockSpec((DIM, DIM), lambda i: (0, 0)),
            pl.BlockSpec((1, DIM), lambda i: (0, 0)),
            pl.BlockSpec((M, DIM, HID), lambda i: (0, 0, 0)),
            pl.BlockSpec((M, HID, 1), lambda i: (0, 0, 0)),
        ],
        out_specs=pl.BlockSpec((M, BN, HID), lambda i: (0, i, 0)),
        out_shape=jax.ShapeDtypeStruct((M, N, HID), jnp.float32),
    )(part, cw1, b1, w2, b2, wn, an)


def _tc_final(part, cw1, b1, w2, b2, batch3, l1, l1b, l2, l2b):
    return pl.pallas_call(
        _tc_final_body,
        grid=(NB,),
        in_specs=[
            pl.BlockSpec((NC, M, BN, HID), lambda i: (0, 0, i, 0)),
            pl.BlockSpec((M, HID, DIM), lambda i: (0, 0, 0)),
            pl.BlockSpec((1, DIM), lambda i: (0, 0)),
            pl.BlockSpec((DIM, DIM), lambda i: (0, 0)),
            pl.BlockSpec((1, DIM), lambda i: (0, 0)),
            pl.BlockSpec((1, 1, BN), lambda i: (i, 0, 0)),
            pl.BlockSpec((DIM, DIM), lambda i: (0, 0)),
            pl.BlockSpec((1, DIM), lambda i: (0, 0)),
            pl.BlockSpec((DIM, OUT), lambda i: (0, 0)),
            pl.BlockSpec((1, OUT), lambda i: (0, 0)),
        ],
        out_specs=pl.BlockSpec((G, OUT), lambda i: (0, 0)),
        out_shape=jax.ShapeDtypeStruct((G, OUT), jnp.float32),
        scratch_shapes=[pltpu.VMEM((G, DIM), jnp.float32)],
    )(part, cw1, b1, w2, b2, batch3, l1, l1b, l2, l2b)


def _prep_mlp(params, l):
    inv = 1.0 / jnp.sqrt(1.0 + 1e-5)
    g = params['bn%d_g' % l] * inv
    w1 = params['mlp%d_w1' % l] * g[None, :]
    b1 = params['mlp%d_b1' % l] * g + params['bn%d_b' % l]
    cw1 = jnp.einsum('mhc,mcd->mhd', params['C%d' % l],
                     w1.reshape(M, CD, DIM))
    return cw1, b1[None, :]


def kernel(x, edge_indices, batch, params):
    src = edge_indices[:, 0, :].astype(jnp.int32)
    dst = edge_indices[:, 1, :].astype(jnp.int32)
    src_adj = src + (jnp.arange(M, dtype=jnp.int32) * N)[:, None]
    src_p = jnp.pad(src_adj.reshape(M, NW, EPT),
                    ((0, 0), (0, 0), (0, EPT_PAD - EPT))).reshape(M, NW, NCHUNK, CH)
    dst_p = jnp.pad(dst.reshape(M, NW, EPT),
                    ((0, 0), (0, 0), (0, EPT_PAD - EPT)),
                    constant_values=N).reshape(M, NW, NCHUNK, CH)

    batch3 = batch.astype(jnp.int32).reshape(NB, 1, BN)
    l1, l1b = params['lin1_w'], params['lin1_b'][None, :]
    l2, l2b = params['lin2_w'], params['lin2_b'][None, :]

    v = _tc_first(x, params['W0'], params['a0'][:, :, None])
    for l in range(3):
        part = _sc_segsum(v.reshape(M * N, HID), src_p, dst_p)
        cw1, b1 = _prep_mlp(params, l)
        w2 = params['mlp%d_w2' % l]
        b2 = params['mlp%d_b2' % l][None, :]
        if l < 2:
            v = _tc_mid(part, cw1, b1, w2, b2,
                        params['W%d' % (l + 1)], params['a%d' % (l + 1)][:, :, None])
        else:
            return _tc_final(part, cw1, b1, w2, b2, batch3, l1, l1b, l2, l2b)


# speedup vs baseline: 6.5203x; 1.0040x over previous
import functools

import jax
import jax.numpy as jnp
from jax import lax
from jax.experimental import pallas as pl
from jax.experimental.pallas import tpu as pltpu
from jax.experimental.pallas import tpu_sc as plsc

N = 10000
D_IN = 128
E = 160000
M = 13
HID = 64
CD = 6
DIM = 64
G = 128
OUT = 10

NC = 2
NS = 16
NW = NC * NS
CH = 128
EPT = E // NW
EPT_PAD = 5120
NCHUNK = EPT_PAD // CH
ROWS_T = 624
ROWS_LAST = 640
ZR = 208
HIDP = 128

BN1 = 2000
BN = 1000
NB = N // BN


NBUF = 8
DEPTH = 4


def _sc_body(v_hbm, src_hbm, dst_hbm, out_hbm,
             src_v, dst_v, rows_v, zeros_v, acc_sh, gsem, ssem):
    c = lax.axis_index("c")
    s = lax.axis_index("s")
    w = c * NS + s

    base = s * ROWS_T

    def _zero_row(r, carry):
        for j in range(HID // 16):
            zeros_v[r, pl.ds(j * 16, 16)] = jnp.zeros((16,), jnp.float32)
        return carry
    lax.fori_loop(0, ZR, _zero_row, 0)

    def _motif(m, carry):
        for z in range(ROWS_T // ZR):
            pltpu.sync_copy(zeros_v, acc_sh.at[pl.ds(base + z * ZR, ZR)])

        @pl.when(s == NS - 1)
        def _():
            pltpu.sync_copy(zeros_v.at[pl.ds(0, ROWS_LAST - ROWS_T)],
                            acc_sh.at[pl.ds(base + ROWS_T, ROWS_LAST - ROWS_T)])
        plsc.subcore_barrier()

        pltpu.sync_copy(src_hbm.at[m, w], src_v)
        pltpu.sync_copy(dst_hbm.at[m, w], dst_v)

        for p in range(DEPTH):
            pltpu.async_copy(v_hbm.at[src_v.at[p]], rows_v.at[p], gsem)

        def _chunk(i, carry2):
            slot = lax.rem(i, NBUF)

            @pl.when(i >= DEPTH)
            def _():
                sl = lax.rem(i - DEPTH, NBUF)
                pltpu.make_async_copy(rows_v.at[sl],
                                      acc_sh.at[dst_v.at[i - DEPTH]], ssem).wait()

            @pl.when(i + DEPTH < NCHUNK)
            def _():
                sl = lax.rem(i + DEPTH, NBUF)
                pltpu.async_copy(v_hbm.at[src_v.at[i + DEPTH]],
                                 rows_v.at[sl], gsem)

            pltpu.make_async_copy(v_hbm.at[src_v.at[i]],
                                  rows_v.at[slot], gsem).wait()
            pltpu.async_copy(rows_v.at[slot], acc_sh.at[dst_v.at[i]],
                             ssem, add=True)
            return carry2
        lax.fori_loop(0, NCHUNK, _chunk, 0)

        for p in range(DEPTH):
            pltpu.make_async_copy(rows_v.at[p],
                                  acc_sh.at[dst_v.at[p]], ssem).wait()

        plsc.subcore_barrier()

        @pl.when(s < NS - 1)
        def _():
            pltpu.sync_copy(acc_sh.at[pl.ds(base, ROWS_T)],
                            out_hbm.at[c, m, pl.ds(base, ROWS_T)])

        @pl.when(s == NS - 1)
        def _():
            pltpu.sync_copy(acc_sh.at[pl.ds(base, ROWS_LAST)],
                            out_hbm.at[c, m, pl.ds(base, ROWS_LAST)])
        return carry
    lax.fori_loop(0, M, _motif, 0)


def _sc_segsum(v_flat, src_p, dst_p):
    mesh = plsc.VectorSubcoreMesh(core_axis_name="c", subcore_axis_name="s",
                                  num_cores=NC, num_subcores=NS)
    kern = functools.partial(
        pl.kernel,
        out_type=jax.ShapeDtypeStruct((NC, M, N, HID), jnp.float32),
        mesh=mesh,
        scratch_types=[
            pltpu.VMEM((NCHUNK, CH), jnp.int32),
            pltpu.VMEM((NCHUNK, CH), jnp.int32),
            pltpu.VMEM((NBUF, CH, HID), jnp.float32),
            pltpu.VMEM((ZR, HID), jnp.float32),
            pltpu.VMEM_SHARED((N + 16, HID), jnp.float32),
            pltpu.SemaphoreType.DMA,
            pltpu.SemaphoreType.DMA,
        ],
        compiler_params=pltpu.CompilerParams(use_tc_tiling_on_sc=False),
    )(_sc_body)
    return kern(v_flat, src_p, dst_p)


def _tc_first_body(x_ref, w_ref, a_ref, v_ref):
    xb = x_ref[...]
    for m in range(M):
        hw = jnp.dot(xb, w_ref[m], preferred_element_type=jnp.float32)
        sc = jnp.tanh(jnp.dot(hw, a_ref[m], preferred_element_type=jnp.float32))
        v_ref[m] = hw * sc


def _h_from_partials(p_ref, cw1_ref, b1_ref, w2_ref, b2_ref):
    acc = jnp.broadcast_to(b1_ref[...], (p_ref.shape[2], DIM))
    for m in range(M):
        t = jnp.tanh(p_ref[0, m] + p_ref[1, m])
        acc = acc + jnp.dot(t, cw1_ref[m], preferred_element_type=jnp.float32)
    h = jnp.maximum(acc, 0.0)
    h = jnp.dot(h, w2_ref[...], preferred_element_type=jnp.float32) + b2_ref[...]
    return jnp.maximum(h, 0.0)


def _tc_mid_body(p_ref, cw1_ref, b1_ref, w2_ref, b2_ref, wn_ref, an_ref, v_ref):
    h = _h_from_partials(p_ref, cw1_ref, b1_ref, w2_ref, b2_ref)
    for m in range(M):
        hw = jnp.dot(h, wn_ref[m], preferred_element_type=jnp.float32)
        sc = jnp.tanh(jnp.dot(hw, an_ref[m], preferred_element_type=jnp.float32))
        v_ref[m] = hw * sc


def _tc_final_body(p_ref, cw1_ref, b1_ref, w2_ref, b2_ref, batch_ref,
                   l1_ref, l1b_ref, l2_ref, l2b_ref, out_ref, hg_acc):
    i = pl.program_id(0)
    h = _h_from_partials(p_ref, cw1_ref, b1_ref, w2_ref, b2_ref)
    b = batch_ref[0]
    oh = (lax.broadcasted_iota(jnp.int32, (G, BN), 0) == b).astype(jnp.float32)
    contrib = jnp.dot(oh, h, preferred_element_type=jnp.float32)

    @pl.when(i == 0)
    def _():
        hg_acc[...] = contrib

    @pl.when(i > 0)
    def _():
        hg_acc[...] = hg_acc[...] + contrib

    @pl.when(i == NB - 1)
    def _():
        hg = jnp.maximum(
            jnp.dot(hg_acc[...], l1_ref[...], preferred_element_type=jnp.float32)
            + l1b_ref[...], 0.0)
        logits = jnp.dot(hg, l2_ref[...], preferred_element_type=jnp.float32) + l2b_ref[...]
        mx = jnp.max(logits, axis=1, keepdims=True)
        z = logits - mx
        lse = jnp.log(jnp.sum(jnp.exp(z), axis=1, keepdims=True))
        out_ref[...] = z - lse


def _tc_first(x, w0, a0):
    nb = N // BN1
    return pl.pallas_call(
        _tc_first_body,
        grid=(nb,),
        in_specs=[
            pl.BlockSpec((BN1, D_IN), lambda i: (i, 0)),
            pl.BlockSpec((M, D_IN, HID), lambda i: (0, 0, 0)),
            pl.BlockSpec((M, HID, 1), lambda i: (0, 0, 0)),
        ],
        out_specs=pl.BlockSpec((M, BN1, HID), lambda i: (0, i, 0)),
        out_shape=jax.ShapeDtypeStruct((M, N, HID), jnp.float32),
    )(x, w0, a0)


def _tc_mid(part, cw1, b1, w2, b2, wn, an):
    return pl.pallas_call(
        _tc_mid_body,
        grid=(NB,),
        in_specs=[
            pl.BlockSpec((NC, M, BN, HID), lambda i: (0, 0, i, 0)),
            pl.BlockSpec((M, HID, DIM), lambda i: (0, 0, 0)),
            pl.BlockSpec((1, DIM), lambda i: (0, 0)),
            pl.BlockSpec((DIM, DIM), lambda i: (0, 0)),
            pl.BlockSpec((1, DIM), lambda i: (0, 0)),
            pl.BlockSpec((M, DIM, HID), lambda i: (0, 0, 0)),
            pl.BlockSpec((M, HID, 1), lambda i: (0, 0, 0)),
        ],
        out_specs=pl.BlockSpec((M, BN, HID), lambda i: (0, i, 0)),
        out_shape=jax.ShapeDtypeStruct((M, N, HID), jnp.float32),
    )(part, cw1, b1, w2, b2, wn, an)


def _tc_final(part, cw1, b1, w2, b2, batch3, l1, l1b, l2, l2b):
    return pl.pallas_call(
        _tc_final_body,
        grid=(NB,),
        in_specs=[
            pl.BlockSpec((NC, M, BN, HID), lambda i: (0, 0, i, 0)),
            pl.BlockSpec((M, HID, DIM), lambda i: (0, 0, 0)),
            pl.BlockSpec((1, DIM), lambda i: (0, 0)),
            pl.BlockSpec((DIM, DIM), lambda i: (0, 0)),
            pl.BlockSpec((1, DIM), lambda i: (0, 0)),
            pl.BlockSpec((1, 1, BN), lambda i: (i, 0, 0)),
            pl.BlockSpec((DIM, DIM), lambda i: (0, 0)),
            pl.BlockSpec((1, DIM), lambda i: (0, 0)),
            pl.BlockSpec((DIM, OUT), lambda i: (0, 0)),
            pl.BlockSpec((1, OUT), lambda i: (0, 0)),
        ],
        out_specs=pl.BlockSpec((G, OUT), lambda i: (0, 0)),
        out_shape=jax.ShapeDtypeStruct((G, OUT), jnp.float32),
        scratch_shapes=[pltpu.VMEM((G, DIM), jnp.float32)],
    )(part, cw1, b1, w2, b2, batch3, l1, l1b, l2, l2b)


def _prep_mlp(params, l):
    inv = 1.0 / jnp.sqrt(1.0 + 1e-5)
    g = params['bn%d_g' % l] * inv
    w1 = params['mlp%d_w1' % l] * g[None, :]
    b1 = params['mlp%d_b1' % l] * g + params['bn%d_b' % l]
    cw1 = jnp.einsum('mhc,mcd->mhd', params['C%d' % l],
                     w1.reshape(M, CD, DIM))
    return cw1, b1[None, :]


def kernel(x, edge_indices, batch, params):
    src = edge_indices[:, 0, :].astype(jnp.int32)
    dst = edge_indices[:, 1, :].astype(jnp.int32)
    src_adj = src + (jnp.arange(M, dtype=jnp.int32) * N)[:, None]
    src_p = jnp.pad(src_adj.reshape(M, NW, EPT),
                    ((0, 0), (0, 0), (0, EPT_PAD - EPT))).reshape(M, NW, NCHUNK, CH)
    dst_p = jnp.pad(dst.reshape(M, NW, EPT),
                    ((0, 0), (0, 0), (0, EPT_PAD - EPT)),
                    constant_values=N).reshape(M, NW, NCHUNK, CH)

    batch3 = batch.astype(jnp.int32).reshape(NB, 1, BN)
    l1, l1b = params['lin1_w'], params['lin1_b'][None, :]
    l2, l2b = params['lin2_w'], params['lin2_b'][None, :]

    v = _tc_first(x, params['W0'], params['a0'][:, :, None])
    for l in range(3):
        part = _sc_segsum(v.reshape(M * N, HID), src_p, dst_p)
        cw1, b1 = _prep_mlp(params, l)
        w2 = params['mlp%d_w2' % l]
        b2 = params['mlp%d_b2' % l][None, :]
        if l < 2:
            v = _tc_mid(part, cw1, b1, w2, b2,
                        params['W%d' % (l + 1)], params['a%d' % (l + 1)][:, :, None])
        else:
            return _tc_final(part, cw1, b1, w2, b2, batch3, l1, l1b, l2, l2b)
